# Initial kernel scaffold; baseline (speedup 1.0000x reference)
import jax
import jax.numpy as jnp
from jax.experimental import pallas as pl


def kernel(values_a, values_b, values_c, values_d, metadata_a, metadata_b, metadata_c, metadata_d, W_meta_qk_outer, W_values_qkv, W_meta_qk_inner, W_out, b_out):
    raise NotImplementedError("write your pallas kernel here")


# speedup vs baseline: 4.5909x; 4.5909x over previous
import jax
import jax.numpy as jnp
from jax.experimental import pallas as pl
from jax.experimental.pallas import tpu as pltpu

_BS = 2
_S = 4
_N = 256
_DV = 512
_DM = 256
_INNER = 512
_H = 8
_DH = _INNER // _H
_G = _BS * _S
_R = _G * _N
_NEG = -1e30


def _prep_kernel(vals_ref, meta_ref, wqkv_ref, winner_ref, wouter_ref,
                 qkv_ref, mproj_ref, bias_ref, means_ref):
    i = pl.program_id(0)
    x = vals_ref[...]
    qkv = jnp.dot(x, wqkv_ref[...], preferred_element_type=jnp.float32)
    qk_part = jnp.clip(qkv[:, : 2 * _INNER], -5.0, 5.0)
    qkv_ref[...] = jnp.concatenate([qk_part, qkv[:, 2 * _INNER:]], axis=1)

    m = meta_ref[...]
    mp = jnp.dot(m, winner_ref[...], preferred_element_type=jnp.float32)
    mproj_ref[...] = jnp.clip(mp, -5.0, 5.0)

    means_ref[pl.ds(i, 1), :] = jnp.mean(m, axis=0, keepdims=True)

    @pl.when(i == _G - 1)
    def _():
        mm = means_ref[...]
        qk = jnp.dot(mm, wouter_ref[...], preferred_element_type=jnp.float32)
        qm = jnp.clip(qk[:, :_INNER], -5.0, 5.0)
        km = jnp.clip(qk[:, _INNER:], -5.0, 5.0)
        dots = jax.lax.dot_general(
            qm, km, (((1,), (1,)), ((), ())),
            preferred_element_type=jnp.float32) * (_INNER ** -0.5)
        v44 = jnp.concatenate([dots[0:4, 0:4], dots[4:8, 4:8]], axis=0)
        mx = jnp.max(v44, axis=1, keepdims=True)
        p = jnp.exp(v44 - mx)
        sm = p / jnp.sum(p, axis=1, keepdims=True)
        rows = jax.lax.broadcasted_iota(jnp.int32, (_G, _S), 0)
        cols = jax.lax.broadcasted_iota(jnp.int32, (_G, _S), 1)
        attn_v = sm + 2.0 * (cols == rows % _S).astype(jnp.float32)
        colv = [attn_v[:, j:j + 1] for j in range(_S)]
        bias_cols = []
        for j in range(_S):
            rank = jnp.zeros((_G, 1), jnp.int32)
            for k in range(_S):
                if k == j:
                    continue
                beats = colv[k] > colv[j]
                if k < j:
                    beats = beats | (colv[k] == colv[j])
                rank = rank + beats.astype(jnp.int32)
            bias_cols.append(jnp.where(rank < _S - 1, 0.0, _NEG))
        key_src = jax.lax.broadcasted_iota(jnp.int32, (_G, _S * _N), 1) // _N
        bias_ref[...] = jnp.where(
            key_src == 0, bias_cols[0],
            jnp.where(key_src == 1, bias_cols[1],
                      jnp.where(key_src == 2, bias_cols[2], bias_cols[3])))


def _attn_kernel(qp_ref, kp_ref, v_ref, qm2_ref, km2_ref, bias_ref,
                 wout_ref, bout_ref, out_ref):
    b = pl.program_id(0)
    s = pl.program_id(1)
    qp = qp_ref[...]
    kp = kp_ref[...]
    v = v_ref[...]
    qm2 = qm2_ref[...]
    km2 = km2_ref[...]
    bias_row = bias_ref[pl.ds(b * _S + s, 1), :]
    acc = bout_ref[...] * jnp.ones((_N, 1), jnp.float32)
    for h in range(_H):
        sl = slice(h * _DH, (h + 1) * _DH)
        qcat = jnp.concatenate([qp[:, sl], qm2[:, sl]], axis=1)
        kcat = jnp.concatenate([kp[:, sl], km2[:, sl]], axis=1)
        scores = jax.lax.dot_general(
            qcat, kcat, (((1,), (1,)), ((), ())),
            preferred_element_type=jnp.float32) * (_DH ** -0.5)
        scores = scores + bias_row
        mx = jnp.max(scores, axis=1, keepdims=True)
        p = jnp.exp(scores - mx)
        attn = p / jnp.sum(p, axis=1, keepdims=True)
        oh = jnp.dot(attn, v[:, sl], preferred_element_type=jnp.float32)
        acc = acc + jnp.dot(oh, wout_ref[sl, :],
                            preferred_element_type=jnp.float32)
    out_ref[...] = acc


def kernel(values_a, values_b, values_c, values_d,
           metadata_a, metadata_b, metadata_c, metadata_d,
           W_meta_qk_outer, W_values_qkv, W_meta_qk_inner, W_out, b_out):
    values = jnp.stack([values_a, values_b, values_c, values_d],
                       axis=1).reshape(_R, _DV)
    meta = jnp.stack([metadata_a, metadata_b, metadata_c, metadata_d],
                     axis=1).reshape(_R, _DM)
    b_out2 = b_out.reshape(1, _DV)

    qkv, mproj, bias = pl.pallas_call(
        _prep_kernel,
        grid=(_G,),
        in_specs=[
            pl.BlockSpec((_N, _DV), lambda i: (i, 0)),
            pl.BlockSpec((_N, _DM), lambda i: (i, 0)),
            pl.BlockSpec((_DV, 3 * _INNER), lambda i: (0, 0)),
            pl.BlockSpec((_DM, 2 * _INNER), lambda i: (0, 0)),
            pl.BlockSpec((_DM, 2 * _INNER), lambda i: (0, 0)),
        ],
        out_specs=[
            pl.BlockSpec((_N, 3 * _INNER), lambda i: (i, 0)),
            pl.BlockSpec((_N, 2 * _INNER), lambda i: (i, 0)),
            pl.BlockSpec((_G, _S * _N), lambda i: (0, 0)),
        ],
        out_shape=[
            jax.ShapeDtypeStruct((_R, 3 * _INNER), jnp.float32),
            jax.ShapeDtypeStruct((_R, 2 * _INNER), jnp.float32),
            jax.ShapeDtypeStruct((_G, _S * _N), jnp.float32),
        ],
        scratch_shapes=[pltpu.VMEM((_G, _DM), jnp.float32)],
    )(values, meta, W_values_qkv, W_meta_qk_inner, W_meta_qk_outer)

    out = pl.pallas_call(
        _attn_kernel,
        grid=(_BS, _S),
        in_specs=[
            pl.BlockSpec((_N, _INNER), lambda b, s: (b * _S + s, 0)),
            pl.BlockSpec((_S * _N, _INNER), lambda b, s: (b, 1)),
            pl.BlockSpec((_S * _N, _INNER), lambda b, s: (b, 2)),
            pl.BlockSpec((_N, _INNER), lambda b, s: (b * _S + s, 0)),
            pl.BlockSpec((_S * _N, _INNER), lambda b, s: (b, 1)),
            pl.BlockSpec((_G, _S * _N), lambda b, s: (0, 0)),
            pl.BlockSpec((_INNER, _DV), lambda b, s: (0, 0)),
            pl.BlockSpec((1, _DV), lambda b, s: (0, 0)),
        ],
        out_specs=pl.BlockSpec((_N, _DV), lambda b, s: (b * _S + s, 0)),
        out_shape=jax.ShapeDtypeStruct((_R, _DV), jnp.float32),
    )(qkv, qkv, qkv, mproj, mproj, bias, W_out, b_out2)

    return out.reshape(_BS, _S, _N, _DV)


# speedup vs baseline: 5.6430x; 1.2292x over previous
import jax
import jax.numpy as jnp
from jax.experimental import pallas as pl
from jax.experimental.pallas import tpu as pltpu

_BS = 2
_S = 4
_N = 256
_DV = 512
_DM = 256
_INNER = 512
_H = 8
_DH = _INNER // _H
_G = _BS * _S
_R = _G * _N
_NEG = -1e30


def _prep_kernel(vals_ref, meta_ref, wqkv_ref, winner_ref, wouter_ref,
                 qkv_ref, mproj_ref, bias_ref, means_ref):
    i = pl.program_id(0)
    x = vals_ref[...]
    qkv = jnp.dot(x, wqkv_ref[...], preferred_element_type=jnp.float32)
    qk_part = jnp.clip(qkv[:, : 2 * _INNER], -5.0, 5.0)
    qkv_ref[...] = jnp.concatenate(
        [qk_part, qkv[:, 2 * _INNER:]], axis=1).astype(jnp.bfloat16)

    m = meta_ref[...]
    mp = jnp.dot(m.astype(jnp.bfloat16), winner_ref[...],
                 preferred_element_type=jnp.float32)
    mproj_ref[...] = jnp.clip(mp, -5.0, 5.0).astype(jnp.bfloat16)

    means_ref[pl.ds(i, 1), :] = jnp.mean(m, axis=0, keepdims=True)

    @pl.when(i == _G - 1)
    def _():
        mm = means_ref[...]
        qk = jnp.dot(mm, wouter_ref[...], preferred_element_type=jnp.float32)
        qm = jnp.clip(qk[:, :_INNER], -5.0, 5.0)
        km = jnp.clip(qk[:, _INNER:], -5.0, 5.0)
        dots = jax.lax.dot_general(
            qm, km, (((1,), (1,)), ((), ())),
            preferred_element_type=jnp.float32) * (_INNER ** -0.5)
        v44 = jnp.concatenate([dots[0:4, 0:4], dots[4:8, 4:8]], axis=0)
        mx = jnp.max(v44, axis=1, keepdims=True)
        p = jnp.exp(v44 - mx)
        sm = p / jnp.sum(p, axis=1, keepdims=True)
        rows = jax.lax.broadcasted_iota(jnp.int32, (_G, _S), 0)
        cols = jax.lax.broadcasted_iota(jnp.int32, (_G, _S), 1)
        attn_v = sm + 2.0 * (cols == rows % _S).astype(jnp.float32)
        colv = [attn_v[:, j:j + 1] for j in range(_S)]
        bias_cols = []
        for j in range(_S):
            rank = jnp.zeros((_G, 1), jnp.int32)
            for k in range(_S):
                if k == j:
                    continue
                beats = colv[k] > colv[j]
                if k < j:
                    beats = beats | (colv[k] == colv[j])
                rank = rank + beats.astype(jnp.int32)
            bias_cols.append(jnp.where(rank < _S - 1, 0.0, _NEG))
        key_src = jax.lax.broadcasted_iota(jnp.int32, (_G, _S * _N), 1) // _N
        bias_ref[...] = jnp.where(
            key_src == 0, bias_cols[0],
            jnp.where(key_src == 1, bias_cols[1],
                      jnp.where(key_src == 2, bias_cols[2], bias_cols[3])))


def _attn_kernel(qp_ref, kp_ref, v_ref, qm2_ref, km2_ref, bias_ref,
                 wout_ref, bout_ref, out_ref):
    b = pl.program_id(0)
    s = pl.program_id(1)
    qp = qp_ref[...]
    kp = kp_ref[...]
    v = v_ref[...]
    qm2 = qm2_ref[...]
    km2 = km2_ref[...]
    bias_row = bias_ref[pl.ds(b * _S + s, 1), :]
    ohs = []
    for h in range(_H):
        sl = slice(h * _DH, (h + 1) * _DH)
        qcat = jnp.concatenate([qp[:, sl], qm2[:, sl]], axis=1)
        kcat = jnp.concatenate([kp[:, sl], km2[:, sl]], axis=1)
        scores = jax.lax.dot_general(
            qcat, kcat, (((1,), (1,)), ((), ())),
            preferred_element_type=jnp.float32) * (_DH ** -0.5)
        scores = scores + bias_row
        mx = jnp.max(scores, axis=1, keepdims=True)
        p = jnp.exp(scores - mx)
        attn = p / jnp.sum(p, axis=1, keepdims=True)
        oh = jnp.dot(attn.astype(jnp.bfloat16), v[:, sl],
                     preferred_element_type=jnp.float32)
        ohs.append(oh.astype(jnp.bfloat16))
    omerged = jnp.concatenate(ohs, axis=1)
    out_ref[...] = jnp.dot(omerged, wout_ref[...],
                           preferred_element_type=jnp.float32) + bout_ref[...]


def kernel(values_a, values_b, values_c, values_d,
           metadata_a, metadata_b, metadata_c, metadata_d,
           W_meta_qk_outer, W_values_qkv, W_meta_qk_inner, W_out, b_out):
    values = jnp.stack([values_a, values_b, values_c, values_d],
                       axis=1).reshape(_R, _DV).astype(jnp.bfloat16)
    meta = jnp.stack([metadata_a, metadata_b, metadata_c, metadata_d],
                     axis=1).reshape(_R, _DM)
    b_out2 = b_out.reshape(1, _DV)
    wqkv = W_values_qkv.astype(jnp.bfloat16)
    winner = W_meta_qk_inner.astype(jnp.bfloat16)
    wout = W_out.astype(jnp.bfloat16)

    qkv, mproj, bias = pl.pallas_call(
        _prep_kernel,
        grid=(_G,),
        in_specs=[
            pl.BlockSpec((_N, _DV), lambda i: (i, 0)),
            pl.BlockSpec((_N, _DM), lambda i: (i, 0)),
            pl.BlockSpec((_DV, 3 * _INNER), lambda i: (0, 0)),
            pl.BlockSpec((_DM, 2 * _INNER), lambda i: (0, 0)),
            pl.BlockSpec((_DM, 2 * _INNER), lambda i: (0, 0)),
        ],
        out_specs=[
            pl.BlockSpec((_N, 3 * _INNER), lambda i: (i, 0)),
            pl.BlockSpec((_N, 2 * _INNER), lambda i: (i, 0)),
            pl.BlockSpec((_G, _S * _N), lambda i: (0, 0)),
        ],
        out_shape=[
            jax.ShapeDtypeStruct((_R, 3 * _INNER), jnp.bfloat16),
            jax.ShapeDtypeStruct((_R, 2 * _INNER), jnp.bfloat16),
            jax.ShapeDtypeStruct((_G, _S * _N), jnp.float32),
        ],
        scratch_shapes=[pltpu.VMEM((_G, _DM), jnp.float32)],
    )(values, meta, wqkv, winner, W_meta_qk_outer)

    out = pl.pallas_call(
        _attn_kernel,
        grid=(_BS, _S),
        in_specs=[
            pl.BlockSpec((_N, _INNER), lambda b, s: (b * _S + s, 0)),
            pl.BlockSpec((_S * _N, _INNER), lambda b, s: (b, 1)),
            pl.BlockSpec((_S * _N, _INNER), lambda b, s: (b, 2)),
            pl.BlockSpec((_N, _INNER), lambda b, s: (b * _S + s, 0)),
            pl.BlockSpec((_S * _N, _INNER), lambda b, s: (b, 1)),
            pl.BlockSpec((_G, _S * _N), lambda b, s: (0, 0)),
            pl.BlockSpec((_INNER, _DV), lambda b, s: (0, 0)),
            pl.BlockSpec((1, _DV), lambda b, s: (0, 0)),
        ],
        out_specs=pl.BlockSpec((_N, _DV), lambda b, s: (b * _S + s, 0)),
        out_shape=jax.ShapeDtypeStruct((_R, _DV), jnp.float32),
    )(qkv, qkv, qkv, mproj, mproj, bias, wout, b_out2)

    return out.reshape(_BS, _S, _N, _DV)


# speedup vs baseline: 6.4953x; 1.1510x over previous
import jax
import jax.numpy as jnp
from jax.experimental import pallas as pl
from jax.experimental.pallas import tpu as pltpu

_BS = 2
_S = 4
_N = 256
_DV = 512
_DM = 256
_INNER = 512
_H = 8
_DH = _INNER // _H
_G = _BS * _S
_R = _G * _N
_L = _S * _N
_NEG = -1e30
_SCALE = _DH ** -0.5


def _prep_kernel(vals_ref, meta_ref, wqkv_ref, winner_ref, wouter_ref,
                 qb_ref, kb_ref, vaug_ref, bias_ref, means_ref):
    i = pl.program_id(0)
    x = vals_ref[...]
    qkv = jnp.dot(x, wqkv_ref[...], preferred_element_type=jnp.float32)
    m = meta_ref[...]
    mp = jnp.dot(m.astype(jnp.bfloat16), winner_ref[...],
                 preferred_element_type=jnp.float32)
    qp = jnp.clip(qkv[:, :_INNER], -5.0, 5.0) * _SCALE
    kp = jnp.clip(qkv[:, _INNER:2 * _INNER], -5.0, 5.0)
    v = qkv[:, 2 * _INNER:]
    qm2 = jnp.clip(mp[:, :_INNER], -5.0, 5.0) * _SCALE
    km2 = jnp.clip(mp[:, _INNER:], -5.0, 5.0)
    ones = jnp.ones((_N, 1), jnp.float32)
    zeros = jnp.zeros((_N, _DH - 1), jnp.float32)
    qbs, kbs, vas = [], [], []
    for h in range(_H):
        sl = slice(h * _DH, (h + 1) * _DH)
        qbs += [qp[:, sl], qm2[:, sl]]
        kbs += [kp[:, sl], km2[:, sl]]
        vas += [v[:, sl], ones, zeros]
    qb_ref[...] = jnp.concatenate(qbs, axis=1).astype(jnp.bfloat16)
    kb_ref[...] = jnp.concatenate(kbs, axis=1).astype(jnp.bfloat16)
    vaug_ref[...] = jnp.concatenate(vas, axis=1).astype(jnp.bfloat16)

    means_ref[pl.ds(i, 1), :] = jnp.mean(m, axis=0, keepdims=True)

    @pl.when(i == _G - 1)
    def _():
        mm = means_ref[...]
        qk = jnp.dot(mm, wouter_ref[...], preferred_element_type=jnp.float32)
        qm = jnp.clip(qk[:, :_INNER], -5.0, 5.0)
        km = jnp.clip(qk[:, _INNER:], -5.0, 5.0)
        dots = jax.lax.dot_general(
            qm, km, (((1,), (1,)), ((), ())),
            preferred_element_type=jnp.float32) * (_INNER ** -0.5)
        v44 = jnp.concatenate([dots[0:4, 0:4], dots[4:8, 4:8]], axis=0)
        mx = jnp.max(v44, axis=1, keepdims=True)
        p = jnp.exp(v44 - mx)
        sm = p / jnp.sum(p, axis=1, keepdims=True)
        rows = jax.lax.broadcasted_iota(jnp.int32, (_G, _S), 0)
        cols = jax.lax.broadcasted_iota(jnp.int32, (_G, _S), 1)
        attn_v = sm + 2.0 * (cols == rows % _S).astype(jnp.float32)
        colv = [attn_v[:, j:j + 1] for j in range(_S)]
        bias_cols = []
        for j in range(_S):
            rank = jnp.zeros((_G, 1), jnp.int32)
            for k in range(_S):
                if k == j:
                    continue
                beats = colv[k] > colv[j]
                if k < j:
                    beats = beats | (colv[k] == colv[j])
                rank = rank + beats.astype(jnp.int32)
            bias_cols.append(jnp.where(rank < _S - 1, 0.0, _NEG))
        key_src = jax.lax.broadcasted_iota(jnp.int32, (_G, _L), 1) // _N
        bias_ref[...] = jnp.where(
            key_src == 0, bias_cols[0],
            jnp.where(key_src == 1, bias_cols[1],
                      jnp.where(key_src == 2, bias_cols[2], bias_cols[3])))


def _attn_kernel(qb_ref, kb_ref, vaug_ref, bias_ref,
                 wout_ref, bout_ref, out_ref):
    b = pl.program_id(0)
    s = pl.program_id(1)
    bias_row = bias_ref[pl.ds(b * _S + s, 1), :]
    ohs = []
    for h in range(_H):
        sl = slice(h * 2 * _DH, (h + 1) * 2 * _DH)
        scores = jax.lax.dot_general(
            qb_ref[:, sl], kb_ref[:, sl], (((1,), (1,)), ((), ())),
            preferred_element_type=jnp.float32)
        s1 = scores + bias_row
        mx = jnp.max(s1, axis=1, keepdims=True)
        p = jnp.exp((s1 - mx).astype(jnp.bfloat16))
        res = jnp.dot(p, vaug_ref[:, sl],
                      preferred_element_type=jnp.float32)
        oh = res[:, :_DH] * (1.0 / res[:, _DH:_DH + 1])
        ohs.append(oh.astype(jnp.bfloat16))
    omerged = jnp.concatenate(ohs, axis=1)
    out_ref[...] = jnp.dot(omerged, wout_ref[...],
                           preferred_element_type=jnp.float32) + bout_ref[...]


def kernel(values_a, values_b, values_c, values_d,
           metadata_a, metadata_b, metadata_c, metadata_d,
           W_meta_qk_outer, W_values_qkv, W_meta_qk_inner, W_out, b_out):
    values = jnp.stack([values_a, values_b, values_c, values_d],
                       axis=1).reshape(_R, _DV).astype(jnp.bfloat16)
    meta = jnp.stack([metadata_a, metadata_b, metadata_c, metadata_d],
                     axis=1).reshape(_R, _DM)
    b_out2 = b_out.reshape(1, _DV)
    wqkv = W_values_qkv.astype(jnp.bfloat16)
    winner = W_meta_qk_inner.astype(jnp.bfloat16)
    wout = W_out.astype(jnp.bfloat16)

    qb, kb, vaug, bias = pl.pallas_call(
        _prep_kernel,
        grid=(_G,),
        in_specs=[
            pl.BlockSpec((_N, _DV), lambda i: (i, 0)),
            pl.BlockSpec((_N, _DM), lambda i: (i, 0)),
            pl.BlockSpec((_DV, 3 * _INNER), lambda i: (0, 0)),
            pl.BlockSpec((_DM, 2 * _INNER), lambda i: (0, 0)),
            pl.BlockSpec((_DM, 2 * _INNER), lambda i: (0, 0)),
        ],
        out_specs=[
            pl.BlockSpec((_N, 2 * _INNER), lambda i: (i, 0)),
            pl.BlockSpec((_N, 2 * _INNER), lambda i: (i, 0)),
            pl.BlockSpec((_N, 2 * _INNER), lambda i: (i, 0)),
            pl.BlockSpec((_G, _L), lambda i: (0, 0)),
        ],
        out_shape=[
            jax.ShapeDtypeStruct((_R, 2 * _INNER), jnp.bfloat16),
            jax.ShapeDtypeStruct((_R, 2 * _INNER), jnp.bfloat16),
            jax.ShapeDtypeStruct((_R, 2 * _INNER), jnp.bfloat16),
            jax.ShapeDtypeStruct((_G, _L), jnp.float32),
        ],
        scratch_shapes=[pltpu.VMEM((_G, _DM), jnp.float32)],
    )(values, meta, wqkv, winner, W_meta_qk_outer)

    out = pl.pallas_call(
        _attn_kernel,
        grid=(_BS, _S),
        in_specs=[
            pl.BlockSpec((_N, 2 * _INNER), lambda b, s: (b * _S + s, 0)),
            pl.BlockSpec((_L, 2 * _INNER), lambda b, s: (b, 0)),
            pl.BlockSpec((_L, 2 * _INNER), lambda b, s: (b, 0)),
            pl.BlockSpec((_G, _L), lambda b, s: (0, 0)),
            pl.BlockSpec((_INNER, _DV), lambda b, s: (0, 0)),
            pl.BlockSpec((1, _DV), lambda b, s: (0, 0)),
        ],
        out_specs=pl.BlockSpec((_N, _DV), lambda b, s: (b * _S + s, 0)),
        out_shape=jax.ShapeDtypeStruct((_R, _DV), jnp.float32),
    )(qb, kb, vaug, bias, wout, b_out2)

    return out.reshape(_BS, _S, _N, _DV)
